# baseline (speedup 1.0000x reference)
import functools

import jax
import jax.numpy as jnp
import numpy as np
from jax import lax
from jax.experimental import pallas as pl
from jax.experimental.pallas import tpu as pltpu
from jax.experimental.pallas import tpu_sc as plsc

P = 9
STRIDE = 16.0
HDIM = 128
NPAD = 5120
GROUP = 16
ENT = 4 * P
NPAIR = 2 * P
ENTP = 48
ROWE = GROUP * NPAIR
NGRP = NPAD // GROUP
NWORK = 32
GPW = NGRP // NWORK


def _prep_body(rois_ref, off_ref, idx_ref, wt_ref):
    H = W = 64
    b_i = rois_ref[0:1, :].astype(jnp.int32)
    x1 = rois_ref[1:2, :]
    y1 = rois_ref[2:3, :]
    x2 = rois_ref[3:4, :]
    y2 = rois_ref[4:5, :]
    cx = (x1 + x2) / 2.0
    cy = (y1 + y2) / 2.0
    w_ = x2 - x1 + 1.0
    h_ = y2 - y1 + 1.0
    for p in range(P):
        ox = off_ref[2 * p:2 * p + 1, :]
        oy = off_ref[2 * p + 1:2 * p + 2, :]
        wp = (cx + ox * w_ * 0.1) / STRIDE
        hp = (cy + oy * h_ * 0.1) / STRIDE
        hl_f = jnp.clip(jnp.floor(hp), 0.0, H - 1.0)
        h_edge = hl_f >= H - 1.0
        hh_f = jnp.where(h_edge, hl_f, hl_f + 1.0)
        hp = jnp.where(h_edge, hl_f, hp)
        wl_f = jnp.clip(jnp.floor(wp), 0.0, W - 1.0)
        w_edge = wl_f >= W - 1.0
        wh_f = jnp.where(w_edge, wl_f, wl_f + 1.0)
        wp = jnp.where(w_edge, wl_f, wp)
        lh = hp - hl_f
        lw = wp - wl_f
        hh = 1.0 - lh
        hw = 1.0 - lw
        hl = hl_f.astype(jnp.int32)
        wl = wl_f.astype(jnp.int32)
        hhi = hh_f.astype(jnp.int32)
        whi = wh_f.astype(jnp.int32)
        base = (p * 2 + b_i) * (H * W)
        idx_ref[2 * p + 0:2 * p + 1, :] = base + hl * W + wl
        idx_ref[2 * p + 1:2 * p + 2, :] = base + hl * W + whi
        e = 4 * p
        wt_ref[e + 0:e + 1, :] = hh * hw
        wt_ref[e + 1:e + 2, :] = lh * hw
        wt_ref[e + 2:e + 3, :] = hh * lw
        wt_ref[e + 3:e + 4, :] = lh * lw


def _prep_call(roisT, offT):
    return pl.pallas_call(
        _prep_body,
        out_shape=(
            jax.ShapeDtypeStruct((NPAIR, NPAD), jnp.int32),
            jax.ShapeDtypeStruct((ENT, NPAD), jnp.float32),
        ),
    )(roisT, offT)


def _g_body(feat_ref, w_ref, g_ref):
    a = lax.dot_general(
        feat_ref[...], w_ref[...],
        dimension_numbers=(((0,), (0,)), ((), ())),
        preferred_element_type=jnp.float32,
    )
    b = lax.bitcast_convert_type(a, jnp.uint32)
    r = (b + 0x7FFF + ((b >> 16) & 1)) >> 16
    r64 = jnp.concatenate([r[64:], r[4032:]], axis=0)
    g_ref[...] = lax.bitcast_convert_type(r | (r64 << 16), jnp.int32)


def _g_call(feat, Wp):
    return pl.pallas_call(
        _g_body,
        grid=(2, P),
        in_specs=[
            pl.BlockSpec((None, 256, 4096), lambda b, p: (b, 0, 0)),
            pl.BlockSpec((None, 256, HDIM), lambda b, p: (p, 0, 0)),
        ],
        out_specs=pl.BlockSpec((None, None, 4096, HDIM),
                               lambda b, p: (p, b, 0, 0)),
        out_shape=jax.ShapeDtypeStruct((P, 2, 4096, HDIM), jnp.int32),
    )(feat, Wp)


_RPW = NPAD // NWORK
_IDXW = _RPW * NPAIR
_WTW = _RPW * ENTP
_CH = ((0, 128, 0), (128, 128, 128), (256, 32, 0))
_SPLAT_DN = lax.GatherDimensionNumbers(
    offset_dims=(), collapsed_slice_dims=(0,), start_index_map=(0,))


def _sc_body(g_hbm, idx_hbm, wt_hbm, out_hbm,
             idx_all, wt_all, rows_a0, rows_a1, rows_a2,
             rows_b0, rows_b1, rows_b2, out_v,
             sem0, sem1, sem2):
    cid = lax.axis_index("c")
    sid = lax.axis_index("s")
    wid = sid * 2 + cid
    rows_a = (rows_a0, rows_a1, rows_a2)
    rows_b = (rows_b0, rows_b1, rows_b2)
    sem = (sem0, sem1, sem2)

    pltpu.sync_copy(idx_hbm.at[pl.ds(wid * _IDXW, _IDXW)], idx_all)
    pltpu.sync_copy(wt_hbm.at[pl.ds(wid * _WTW, _WTW)], wt_all)

    def dmas(g, b):
        out = []
        for (src, ln, dst) in _CH:
            dref = rows_a[b] if src < 256 else rows_b[b]
            out.append((
                g_hbm.at[idx_all.at[pl.ds(g * ROWE + src, ln)]],
                dref.at[pl.ds(dst, ln)],
                sem[b],
            ))
        return out

    def fetch(g, b):
        for args in dmas(g, b):
            pltpu.async_copy(*args)

    def drain(g, b):
        for args in dmas(g, b):
            pltpu.make_async_copy(*args).wait()

    def accum_roi(b, g, r, row_at):
        acc = tuple(jnp.zeros((16,), jnp.float32) for _ in range(16))
        wbase = (g * GROUP + r) * ENTP
        for q in range(3):
            w16 = wt_all[pl.ds(wbase + q * 16, 16)]
            for kk in range(8):
                j = q * 8 + kk
                if j >= NPAIR:
                    break
                wA = lax.gather(
                    w16, jnp.full((16, 1), 2 * kk, jnp.int32), _SPLAT_DN,
                    slice_sizes=(1,),
                    mode=lax.GatherScatterMode.PROMISE_IN_BOUNDS)
                wB = lax.gather(
                    w16, jnp.full((16, 1), 2 * kk + 1, jnp.int32), _SPLAT_DN,
                    slice_sizes=(1,),
                    mode=lax.GatherScatterMode.PROMISE_IN_BOUNDS)
                ref, row = row_at(r * NPAIR + j)
                new = list(acc)
                for m in range(8):
                    wi = ref[row, pl.ds(m * 16, 16)]
                    lo = plsc.bitcast(wi << 16, jnp.float32)
                    hi = plsc.bitcast(wi & (-65536), jnp.float32)
                    new[m] = new[m] + wA * lo
                    new[8 + m] = new[8 + m] + wB * hi
                acc = tuple(new)
        for c in range(8):
            out_v[r, pl.ds(c * 16, 16)] = acc[c] + acc[8 + c]

    def compute(b, g):
        def roi_body(r, carry2):
            accum_roi(b, g, r, lambda e: (rows_a[b], e))
            return carry2

        lax.fori_loop(0, GROUP - 2, roi_body, 0)
        accum_roi(b, g, GROUP - 2,
                  lambda e: (rows_a[b], e) if e < 256 else (rows_b[b], e - 256))
        accum_roi(b, g, GROUP - 1, lambda e: (rows_b[b], e - 256))
        grp = wid * GPW + g
        pltpu.sync_copy(out_v, out_hbm.at[pl.ds(grp * GROUP, GROUP)])

    fetch(0, 0)
    fetch(1, 1)

    def tri_body(g3, carry):
        for u in range(3):
            g = g3 * 3 + u

            @pl.when(g + 2 < GPW)
            def _():
                fetch(g + 2, (u + 2) % 3)

            drain(g, u)
            compute(u, g)
        return carry

    lax.fori_loop(0, (GPW // 3) * 3 // 3, tri_body, 0)
    for g in range((GPW // 3) * 3, GPW):
        drain(g, g % 3)
        compute(g % 3, g)


def _sc_call(Gf, idx, wt):
    mesh = plsc.VectorSubcoreMesh(core_axis_name="c", subcore_axis_name="s")
    fn = functools.partial(
        pl.kernel,
        out_type=jax.ShapeDtypeStruct((NPAD, HDIM), jnp.float32),
        mesh=mesh,
        compiler_params=pltpu.CompilerParams(needs_layout_passes=False),
        scratch_types=[
            pltpu.VMEM((_IDXW,), jnp.int32),
            pltpu.VMEM((_WTW,), jnp.float32),
            pltpu.VMEM((256, HDIM), jnp.int32),
            pltpu.VMEM((256, HDIM), jnp.int32),
            pltpu.VMEM((256, HDIM), jnp.int32),
            pltpu.VMEM((32, HDIM), jnp.int32),
            pltpu.VMEM((32, HDIM), jnp.int32),
            pltpu.VMEM((32, HDIM), jnp.int32),
            pltpu.VMEM((GROUP, HDIM), jnp.float32),
            pltpu.SemaphoreType.DMA,
            pltpu.SemaphoreType.DMA,
            pltpu.SemaphoreType.DMA,
        ],
    )(_sc_body)
    return fn(Gf, idx, wt)


def _box_body(rois_ref, off_ref, loc_ref, box_ref):
    x1 = rois_ref[1:2, :]
    y1 = rois_ref[2:3, :]
    x2 = rois_ref[3:4, :]
    y2 = rois_ref[4:5, :]
    cx = (x1 + x2) / 2.0
    cy = (y1 + y2) / 2.0
    w_ = x2 - x1 + 1.0
    h_ = y2 - y1 + 1.0
    xmin = xmax = ymin = ymax = None
    for p in range(P):
        ox = off_ref[2 * p:2 * p + 1, :]
        oy = off_ref[2 * p + 1:2 * p + 2, :]
        lx = loc_ref[2 * p:2 * p + 1, :]
        ly = loc_ref[2 * p + 1:2 * p + 2, :]
        sx = (cx + ox * w_ * 0.1) + lx * w_ * 0.5
        sy = (cy + oy * h_ * 0.1) + ly * h_ * 0.5
        if p == 0:
            xmin = xmax = sx
            ymin = ymax = sy
        else:
            xmin = jnp.minimum(xmin, sx)
            xmax = jnp.maximum(xmax, sx)
            ymin = jnp.minimum(ymin, sy)
            ymax = jnp.maximum(ymax, sy)
    box_ref[0:1, :] = xmin
    box_ref[1:2, :] = ymin
    box_ref[2:3, :] = xmax
    box_ref[3:4, :] = ymax


def _box_call(roisT, offT, locT):
    return pl.pallas_call(
        _box_body,
        out_shape=jax.ShapeDtypeStruct((4, NPAD), jnp.float32),
    )(roisT, offT, locT)


def kernel(feat_map, rois, offset, W_cls, W_loc):
    B, C, H, W = feat_map.shape
    N = rois.shape[0]
    roisT = jnp.zeros((5, NPAD), jnp.float32).at[:, :N].set(rois.T)
    offT = jnp.zeros((2 * P, NPAD), jnp.float32).at[:, :N].set(offset.T)

    idxT, wtT = _prep_call(roisT, offT)
    idx = idxT.T.reshape(NPAD * NPAIR)
    wt = jnp.pad(wtT.T, ((0, 0), (0, ENTP - ENT))).reshape(NPAD * ENTP)

    Wcat = jnp.concatenate([W_cls, W_loc], axis=1)
    Wp = Wcat.reshape(P, C, 99)
    Wp = jnp.pad(Wp, ((0, 0), (0, 0), (0, HDIM - 99)))
    Wp = Wp.astype(jnp.bfloat16)
    feat = feat_map.reshape(B, C, H * W).astype(jnp.bfloat16)

    G = _g_call(feat, Wp)
    Gf = G.reshape(P * B * H * W, HDIM)

    out = _sc_call(Gf, idx, wt)

    pred_cls = out[:N, :81]
    locT = out[:, 81:99].T
    boxT = _box_call(roisT, offT, locT)
    boxes = boxT[:, :N].T
    return pred_cls, boxes


# speedup vs baseline: 1.0120x; 1.0120x over previous
import functools

import jax
import jax.numpy as jnp
import numpy as np
from jax import lax
from jax.experimental import pallas as pl
from jax.experimental.pallas import tpu as pltpu
from jax.experimental.pallas import tpu_sc as plsc

P = 9
STRIDE = 16.0
HDIM = 128
NPAD = 5120
GROUP = 16
ENT = 4 * P
NPAIR = 2 * P
ENTP = 48
ROWE = GROUP * NPAIR
NGRP = NPAD // GROUP
NWORK = 32
GPW = NGRP // NWORK


def _prep_body(rois_ref, off_ref, idx_ref, wt_ref):
    H = W = 64
    wt_ref[ENT:ENTP, :] = jnp.zeros((ENTP - ENT, NPAD), jnp.float32)
    b_i = rois_ref[0:1, :].astype(jnp.int32)
    x1 = rois_ref[1:2, :]
    y1 = rois_ref[2:3, :]
    x2 = rois_ref[3:4, :]
    y2 = rois_ref[4:5, :]
    cx = (x1 + x2) / 2.0
    cy = (y1 + y2) / 2.0
    w_ = x2 - x1 + 1.0
    h_ = y2 - y1 + 1.0
    for p in range(P):
        ox = off_ref[2 * p:2 * p + 1, :]
        oy = off_ref[2 * p + 1:2 * p + 2, :]
        wp = (cx + ox * w_ * 0.1) / STRIDE
        hp = (cy + oy * h_ * 0.1) / STRIDE
        hl_f = jnp.clip(jnp.floor(hp), 0.0, H - 1.0)
        h_edge = hl_f >= H - 1.0
        hh_f = jnp.where(h_edge, hl_f, hl_f + 1.0)
        hp = jnp.where(h_edge, hl_f, hp)
        wl_f = jnp.clip(jnp.floor(wp), 0.0, W - 1.0)
        w_edge = wl_f >= W - 1.0
        wh_f = jnp.where(w_edge, wl_f, wl_f + 1.0)
        wp = jnp.where(w_edge, wl_f, wp)
        lh = hp - hl_f
        lw = wp - wl_f
        hh = 1.0 - lh
        hw = 1.0 - lw
        hl = hl_f.astype(jnp.int32)
        wl = wl_f.astype(jnp.int32)
        hhi = hh_f.astype(jnp.int32)
        whi = wh_f.astype(jnp.int32)
        base = (p * 2 + b_i) * (H * W)
        idx_ref[2 * p + 0:2 * p + 1, :] = base + hl * W + wl
        idx_ref[2 * p + 1:2 * p + 2, :] = base + hl * W + whi
        e = 4 * p
        wt_ref[e + 0:e + 1, :] = hh * hw
        wt_ref[e + 1:e + 2, :] = lh * hw
        wt_ref[e + 2:e + 3, :] = hh * lw
        wt_ref[e + 3:e + 4, :] = lh * lw


def _prep_call(roisT, offT):
    return pl.pallas_call(
        _prep_body,
        out_shape=(
            jax.ShapeDtypeStruct((NPAIR, NPAD), jnp.int32),
            jax.ShapeDtypeStruct((ENTP, NPAD), jnp.float32),
        ),
    )(roisT, offT)


def _g_body(feat_ref, w_ref, g_ref):
    a = lax.dot_general(
        feat_ref[...], w_ref[...],
        dimension_numbers=(((0,), (0,)), ((), ())),
        preferred_element_type=jnp.float32,
    )
    b = lax.bitcast_convert_type(a, jnp.uint32)
    r = (b + 0x7FFF + ((b >> 16) & 1)) >> 16
    r64 = jnp.concatenate([r[64:], r[4032:]], axis=0)
    g_ref[...] = lax.bitcast_convert_type(r | (r64 << 16), jnp.int32)


def _g_call(feat, Wp):
    return pl.pallas_call(
        _g_body,
        grid=(2, P),
        in_specs=[
            pl.BlockSpec((None, 256, 4096), lambda b, p: (b, 0, 0)),
            pl.BlockSpec((None, 256, HDIM), lambda b, p: (p, 0, 0)),
        ],
        out_specs=pl.BlockSpec((None, None, 4096, HDIM),
                               lambda b, p: (p, b, 0, 0)),
        out_shape=jax.ShapeDtypeStruct((P, 2, 4096, HDIM), jnp.int32),
    )(feat, Wp)


_RPW = NPAD // NWORK
_IDXW = _RPW * NPAIR
_WTW = _RPW * ENTP
_CH = ((0, 128, 0), (128, 128, 128), (256, 32, 0))
_SPLAT_DN = lax.GatherDimensionNumbers(
    offset_dims=(), collapsed_slice_dims=(0,), start_index_map=(0,))


def _sc_body(g_hbm, idx_hbm, wt_hbm, out_hbm,
             idx_all, wt_all, rows_a0, rows_a1, rows_a2,
             rows_b0, rows_b1, rows_b2, out_v,
             sem0, sem1, sem2):
    cid = lax.axis_index("c")
    sid = lax.axis_index("s")
    wid = sid * 2 + cid
    rows_a = (rows_a0, rows_a1, rows_a2)
    rows_b = (rows_b0, rows_b1, rows_b2)
    sem = (sem0, sem1, sem2)

    pltpu.sync_copy(idx_hbm.at[pl.ds(wid * _IDXW, _IDXW)], idx_all)
    pltpu.sync_copy(wt_hbm.at[pl.ds(wid * _WTW, _WTW)], wt_all)

    def dmas(g, b):
        out = []
        for (src, ln, dst) in _CH:
            dref = rows_a[b] if src < 256 else rows_b[b]
            out.append((
                g_hbm.at[idx_all.at[pl.ds(g * ROWE + src, ln)]],
                dref.at[pl.ds(dst, ln)],
                sem[b],
            ))
        return out

    def fetch(g, b):
        for args in dmas(g, b):
            pltpu.async_copy(*args)

    def drain(g, b):
        for args in dmas(g, b):
            pltpu.make_async_copy(*args).wait()

    def accum_roi(b, g, r, row_at):
        acc = tuple(jnp.zeros((16,), jnp.float32) for _ in range(16))
        wbase = (g * GROUP + r) * ENTP
        for q in range(3):
            w16 = wt_all[pl.ds(wbase + q * 16, 16)]
            for kk in range(8):
                j = q * 8 + kk
                if j >= NPAIR:
                    break
                wA = lax.gather(
                    w16, jnp.full((16, 1), 2 * kk, jnp.int32), _SPLAT_DN,
                    slice_sizes=(1,),
                    mode=lax.GatherScatterMode.PROMISE_IN_BOUNDS)
                wB = lax.gather(
                    w16, jnp.full((16, 1), 2 * kk + 1, jnp.int32), _SPLAT_DN,
                    slice_sizes=(1,),
                    mode=lax.GatherScatterMode.PROMISE_IN_BOUNDS)
                ref, row = row_at(r * NPAIR + j)
                new = list(acc)
                for m in range(8):
                    wi = ref[row, pl.ds(m * 16, 16)]
                    lo = plsc.bitcast(wi << 16, jnp.float32)
                    hi = plsc.bitcast(wi & (-65536), jnp.float32)
                    new[m] = new[m] + wA * lo
                    new[8 + m] = new[8 + m] + wB * hi
                acc = tuple(new)
        for c in range(8):
            out_v[r, pl.ds(c * 16, 16)] = acc[c] + acc[8 + c]

    def compute(b, g):
        def roi_body(r, carry2):
            accum_roi(b, g, r, lambda e: (rows_a[b], e))
            return carry2

        lax.fori_loop(0, GROUP - 2, roi_body, 0)
        accum_roi(b, g, GROUP - 2,
                  lambda e: (rows_a[b], e) if e < 256 else (rows_b[b], e - 256))
        accum_roi(b, g, GROUP - 1, lambda e: (rows_b[b], e - 256))
        grp = wid * GPW + g
        pltpu.sync_copy(out_v, out_hbm.at[pl.ds(grp * GROUP, GROUP)])

    fetch(0, 0)
    fetch(1, 1)

    def tri_body(g3, carry):
        for u in range(3):
            g = g3 * 3 + u

            @pl.when(g + 2 < GPW)
            def _():
                fetch(g + 2, (u + 2) % 3)

            drain(g, u)
            compute(u, g)
        return carry

    lax.fori_loop(0, (GPW // 3) * 3 // 3, tri_body, 0)
    for g in range((GPW // 3) * 3, GPW):
        drain(g, g % 3)
        compute(g % 3, g)


def _sc_call(Gf, idx, wt):
    mesh = plsc.VectorSubcoreMesh(core_axis_name="c", subcore_axis_name="s")
    fn = functools.partial(
        pl.kernel,
        out_type=jax.ShapeDtypeStruct((NPAD, HDIM), jnp.float32),
        mesh=mesh,
        compiler_params=pltpu.CompilerParams(needs_layout_passes=False),
        scratch_types=[
            pltpu.VMEM((_IDXW,), jnp.int32),
            pltpu.VMEM((_WTW,), jnp.float32),
            pltpu.VMEM((256, HDIM), jnp.int32),
            pltpu.VMEM((256, HDIM), jnp.int32),
            pltpu.VMEM((256, HDIM), jnp.int32),
            pltpu.VMEM((32, HDIM), jnp.int32),
            pltpu.VMEM((32, HDIM), jnp.int32),
            pltpu.VMEM((32, HDIM), jnp.int32),
            pltpu.VMEM((GROUP, HDIM), jnp.float32),
            pltpu.SemaphoreType.DMA,
            pltpu.SemaphoreType.DMA,
            pltpu.SemaphoreType.DMA,
        ],
    )(_sc_body)
    return fn(Gf, idx, wt)


def _box_body(rois_ref, off_ref, loc_ref, box_ref):
    x1 = rois_ref[1:2, :]
    y1 = rois_ref[2:3, :]
    x2 = rois_ref[3:4, :]
    y2 = rois_ref[4:5, :]
    cx = (x1 + x2) / 2.0
    cy = (y1 + y2) / 2.0
    w_ = x2 - x1 + 1.0
    h_ = y2 - y1 + 1.0
    xmin = xmax = ymin = ymax = None
    for p in range(P):
        ox = off_ref[2 * p:2 * p + 1, :]
        oy = off_ref[2 * p + 1:2 * p + 2, :]
        lx = loc_ref[2 * p:2 * p + 1, :]
        ly = loc_ref[2 * p + 1:2 * p + 2, :]
        sx = (cx + ox * w_ * 0.1) + lx * w_ * 0.5
        sy = (cy + oy * h_ * 0.1) + ly * h_ * 0.5
        if p == 0:
            xmin = xmax = sx
            ymin = ymax = sy
        else:
            xmin = jnp.minimum(xmin, sx)
            xmax = jnp.maximum(xmax, sx)
            ymin = jnp.minimum(ymin, sy)
            ymax = jnp.maximum(ymax, sy)
    box_ref[0:1, :] = xmin
    box_ref[1:2, :] = ymin
    box_ref[2:3, :] = xmax
    box_ref[3:4, :] = ymax


def _box_call(roisT, offT, locT):
    return pl.pallas_call(
        _box_body,
        out_shape=jax.ShapeDtypeStruct((4, NPAD), jnp.float32),
    )(roisT, offT, locT)


def kernel(feat_map, rois, offset, W_cls, W_loc):
    B, C, H, W = feat_map.shape
    N = rois.shape[0]
    roisT = jnp.zeros((5, NPAD), jnp.float32).at[:, :N].set(rois.T)
    offT = jnp.zeros((2 * P, NPAD), jnp.float32).at[:, :N].set(offset.T)

    idxT, wtT = _prep_call(roisT, offT)
    idx = idxT.T.reshape(NPAD * NPAIR)
    wt = wtT.T.reshape(NPAD * ENTP)

    Wcat = jnp.concatenate([W_cls, W_loc], axis=1)
    Wp = Wcat.reshape(P, C, 99)
    Wp = jnp.pad(Wp, ((0, 0), (0, 0), (0, HDIM - 99)))
    Wp = Wp.astype(jnp.bfloat16)
    feat = feat_map.reshape(B, C, H * W).astype(jnp.bfloat16)

    G = _g_call(feat, Wp)
    Gf = G.reshape(P * B * H * W, HDIM)

    out = _sc_call(Gf, idx, wt)

    pred_cls = out[:N, :81]
    locT = out[:, 81:99].T
    boxT = _box_call(roisT, offT, locT)
    boxes = boxT[:, :N].T
    return pred_cls, boxes


# speedup vs baseline: 1.0126x; 1.0006x over previous
import functools

import jax
import jax.numpy as jnp
from jax import lax
from jax.experimental import pallas as pl
from jax.experimental.pallas import tpu as pltpu
from jax.experimental.pallas import tpu_sc as plsc

P = 9
STRIDE = 16.0
HDIM = 128
NPAD = 5120
GROUP = 16
ENT = 4 * P
NPAIR = 2 * P
ENTP = 48
ROWE = GROUP * NPAIR
NGRP = NPAD // GROUP
NWORK = 32
GPW = NGRP // NWORK


def _prep_body(rois_ref, off_ref, idx_ref, wt_ref):
    H = W = 64
    wt_ref[ENT:ENTP, :] = jnp.zeros((ENTP - ENT, NPAD), jnp.float32)
    b_i = rois_ref[0:1, :].astype(jnp.int32)
    x1 = rois_ref[1:2, :]
    y1 = rois_ref[2:3, :]
    x2 = rois_ref[3:4, :]
    y2 = rois_ref[4:5, :]
    cx = (x1 + x2) / 2.0
    cy = (y1 + y2) / 2.0
    w_ = x2 - x1 + 1.0
    h_ = y2 - y1 + 1.0
    for p in range(P):
        ox = off_ref[2 * p:2 * p + 1, :]
        oy = off_ref[2 * p + 1:2 * p + 2, :]
        wp = (cx + ox * w_ * 0.1) / STRIDE
        hp = (cy + oy * h_ * 0.1) / STRIDE
        hl_f = jnp.clip(jnp.floor(hp), 0.0, H - 1.0)
        h_edge = hl_f >= H - 1.0
        hh_f = jnp.where(h_edge, hl_f, hl_f + 1.0)
        hp = jnp.where(h_edge, hl_f, hp)
        wl_f = jnp.clip(jnp.floor(wp), 0.0, W - 1.0)
        w_edge = wl_f >= W - 1.0
        wh_f = jnp.where(w_edge, wl_f, wl_f + 1.0)
        wp = jnp.where(w_edge, wl_f, wp)
        lh = hp - hl_f
        lw = wp - wl_f
        hh = 1.0 - lh
        hw = 1.0 - lw
        hl = hl_f.astype(jnp.int32)
        wl = wl_f.astype(jnp.int32)
        hhi = hh_f.astype(jnp.int32)
        whi = wh_f.astype(jnp.int32)
        base = (p * 2 + b_i) * (H * W)
        idx_ref[2 * p + 0:2 * p + 1, :] = base + hl * W + wl
        idx_ref[2 * p + 1:2 * p + 2, :] = base + hl * W + whi
        e = 4 * p
        wt_ref[e + 0:e + 1, :] = hh * hw
        wt_ref[e + 1:e + 2, :] = lh * hw
        wt_ref[e + 2:e + 3, :] = hh * lw
        wt_ref[e + 3:e + 4, :] = lh * lw


def _prep_call(roisT, offT):
    return pl.pallas_call(
        _prep_body,
        out_shape=(
            jax.ShapeDtypeStruct((NPAIR, NPAD), jnp.int32),
            jax.ShapeDtypeStruct((ENTP, NPAD), jnp.float32),
        ),
    )(roisT, offT)


def _g_body(feat_ref, w_ref, g_ref):
    a = lax.dot_general(
        feat_ref[...], w_ref[...],
        dimension_numbers=(((0,), (0,)), ((), ())),
        preferred_element_type=jnp.float32,
    )
    b = lax.bitcast_convert_type(a, jnp.uint32)
    r = (b + 0x7FFF + ((b >> 16) & 1)) >> 16
    r64 = jnp.concatenate([r[64:], r[4032:]], axis=0)
    g_ref[...] = lax.bitcast_convert_type(r | (r64 << 16), jnp.int32)


def _g_call(feat, Wp):
    return pl.pallas_call(
        _g_body,
        grid=(2, P),
        in_specs=[
            pl.BlockSpec((None, 256, 4096), lambda b, p: (b, 0, 0)),
            pl.BlockSpec((None, 256, HDIM), lambda b, p: (p, 0, 0)),
        ],
        out_specs=pl.BlockSpec((None, None, 4096, HDIM),
                               lambda b, p: (p, b, 0, 0)),
        out_shape=jax.ShapeDtypeStruct((P, 2, 4096, HDIM), jnp.int32),
    )(feat, Wp)


_RPW = NPAD // NWORK
_IDXW = _RPW * NPAIR
_WTW = _RPW * ENTP
_CH = ((0, 128, 0), (128, 128, 128), (256, 32, 0))
_SPLAT_DN = lax.GatherDimensionNumbers(
    offset_dims=(), collapsed_slice_dims=(0,), start_index_map=(0,))


def _sc_body(g_hbm, idx_hbm, wt_hbm, out_hbm,
             idx_all, wt_all, rows_a0, rows_a1, rows_a2,
             rows_b0, rows_b1, rows_b2, out_v,
             sem0, sem1, sem2):
    cid = lax.axis_index("c")
    sid = lax.axis_index("s")
    wid = sid * 2 + cid
    rows_a = (rows_a0, rows_a1, rows_a2)
    rows_b = (rows_b0, rows_b1, rows_b2)
    sem = (sem0, sem1, sem2)

    pltpu.sync_copy(idx_hbm.at[pl.ds(wid * _IDXW, _IDXW)], idx_all)
    pltpu.sync_copy(wt_hbm.at[pl.ds(wid * _WTW, _WTW)], wt_all)

    def dmas(g, b):
        out = []
        for (src, ln, dst) in _CH:
            dref = rows_a[b] if src < 256 else rows_b[b]
            out.append((
                g_hbm.at[idx_all.at[pl.ds(g * ROWE + src, ln)]],
                dref.at[pl.ds(dst, ln)],
                sem[b],
            ))
        return out

    def fetch(g, b):
        for args in dmas(g, b):
            pltpu.async_copy(*args)

    def drain(g, b):
        for args in dmas(g, b):
            pltpu.make_async_copy(*args).wait()

    def accum_roi(b, g, r, row_at):
        acc = tuple(jnp.zeros((16,), jnp.float32) for _ in range(16))
        wbase = (g * GROUP + r) * ENTP
        for q in range(3):
            w16 = wt_all[pl.ds(wbase + q * 16, 16)]
            for kk in range(8):
                j = q * 8 + kk
                if j >= NPAIR:
                    break
                wA = lax.gather(
                    w16, jnp.full((16, 1), 2 * kk, jnp.int32), _SPLAT_DN,
                    slice_sizes=(1,),
                    mode=lax.GatherScatterMode.PROMISE_IN_BOUNDS)
                wB = lax.gather(
                    w16, jnp.full((16, 1), 2 * kk + 1, jnp.int32), _SPLAT_DN,
                    slice_sizes=(1,),
                    mode=lax.GatherScatterMode.PROMISE_IN_BOUNDS)
                ref, row = row_at(r * NPAIR + j)
                new = list(acc)
                for m in range(8):
                    wi = ref[row, pl.ds(m * 16, 16)]
                    lo = plsc.bitcast(wi << 16, jnp.float32)
                    hi = plsc.bitcast(wi & (-65536), jnp.float32)
                    new[m] = new[m] + wA * lo
                    new[8 + m] = new[8 + m] + wB * hi
                acc = tuple(new)
        for c in range(8):
            out_v[r, pl.ds(c * 16, 16)] = acc[c] + acc[8 + c]

    def compute(b, g):
        def roi_body(r, carry2):
            accum_roi(b, g, r, lambda e: (rows_a[b], e))
            return carry2

        lax.fori_loop(0, GROUP - 2, roi_body, 0)
        accum_roi(b, g, GROUP - 2,
                  lambda e: (rows_a[b], e) if e < 256 else (rows_b[b], e - 256))
        accum_roi(b, g, GROUP - 1, lambda e: (rows_b[b], e - 256))
        grp = wid * GPW + g
        pltpu.sync_copy(out_v, out_hbm.at[pl.ds(grp * GROUP, GROUP)])

    fetch(0, 0)
    fetch(1, 1)

    def tri_body(g3, carry):
        for u in range(3):
            g = g3 * 3 + u

            @pl.when(g + 2 < GPW)
            def _():
                fetch(g + 2, (u + 2) % 3)

            drain(g, u)
            compute(u, g)
        return carry

    lax.fori_loop(0, (GPW // 3) * 3 // 3, tri_body, 0)
    for g in range((GPW // 3) * 3, GPW):
        drain(g, g % 3)
        compute(g % 3, g)


def _sc_call(Gf, idx, wt):
    mesh = plsc.VectorSubcoreMesh(core_axis_name="c", subcore_axis_name="s")
    fn = functools.partial(
        pl.kernel,
        out_type=jax.ShapeDtypeStruct((NPAD, HDIM), jnp.float32),
        mesh=mesh,
        compiler_params=pltpu.CompilerParams(needs_layout_passes=False),
        scratch_types=[
            pltpu.VMEM((_IDXW,), jnp.int32),
            pltpu.VMEM((_WTW,), jnp.float32),
            pltpu.VMEM((256, HDIM), jnp.int32),
            pltpu.VMEM((256, HDIM), jnp.int32),
            pltpu.VMEM((256, HDIM), jnp.int32),
            pltpu.VMEM((32, HDIM), jnp.int32),
            pltpu.VMEM((32, HDIM), jnp.int32),
            pltpu.VMEM((32, HDIM), jnp.int32),
            pltpu.VMEM((GROUP, HDIM), jnp.float32),
            pltpu.SemaphoreType.DMA,
            pltpu.SemaphoreType.DMA,
            pltpu.SemaphoreType.DMA,
        ],
    )(_sc_body)
    return fn(Gf, idx, wt)


def _box_body(rois_ref, off_ref, loc_ref, box_ref):
    x1 = rois_ref[1:2, :]
    y1 = rois_ref[2:3, :]
    x2 = rois_ref[3:4, :]
    y2 = rois_ref[4:5, :]
    cx = (x1 + x2) / 2.0
    cy = (y1 + y2) / 2.0
    w_ = x2 - x1 + 1.0
    h_ = y2 - y1 + 1.0
    xmin = xmax = ymin = ymax = None
    for p in range(P):
        ox = off_ref[2 * p:2 * p + 1, :]
        oy = off_ref[2 * p + 1:2 * p + 2, :]
        lx = loc_ref[2 * p:2 * p + 1, :]
        ly = loc_ref[2 * p + 1:2 * p + 2, :]
        sx = (cx + ox * w_ * 0.1) + lx * w_ * 0.5
        sy = (cy + oy * h_ * 0.1) + ly * h_ * 0.5
        if p == 0:
            xmin = xmax = sx
            ymin = ymax = sy
        else:
            xmin = jnp.minimum(xmin, sx)
            xmax = jnp.maximum(xmax, sx)
            ymin = jnp.minimum(ymin, sy)
            ymax = jnp.maximum(ymax, sy)
    box_ref[0:1, :] = xmin
    box_ref[1:2, :] = ymin
    box_ref[2:3, :] = xmax
    box_ref[3:4, :] = ymax


def _box_call(roisT, offT, locT):
    return pl.pallas_call(
        _box_body,
        out_shape=jax.ShapeDtypeStruct((4, NPAD), jnp.float32),
    )(roisT, offT, locT)


def kernel(feat_map, rois, offset, W_cls, W_loc):
    B, C, H, W = feat_map.shape
    N = rois.shape[0]
    roisT = jnp.zeros((5, NPAD), jnp.float32).at[:, :N].set(rois.T)
    offT = jnp.zeros((2 * P, NPAD), jnp.float32).at[:, :N].set(offset.T)

    idxT, wtT = _prep_call(roisT, offT)
    idx = idxT.T.reshape(NPAD * NPAIR)
    wt = wtT.T.reshape(NPAD * ENTP)

    Wcat = jnp.concatenate([W_cls, W_loc], axis=1)
    Wp = Wcat.reshape(P, C, 99)
    Wp = jnp.pad(Wp, ((0, 0), (0, 0), (0, HDIM - 99)))
    Wp = Wp.astype(jnp.bfloat16)
    feat = feat_map.reshape(B, C, H * W).astype(jnp.bfloat16)

    G = _g_call(feat, Wp)
    Gf = G.reshape(P * B * H * W, HDIM)

    out = _sc_call(Gf, idx, wt)

    pred_cls = out[:N, :81]
    locT = out[:, 81:99].T
    boxT = _box_call(roisT, offT, locT)
    boxes = boxT[:, :N].T
    return pred_cls, boxes
